# baseline (speedup 1.0000x reference)
import functools

import jax
import jax.numpy as jnp
from jax import lax
from jax.experimental import pallas as pl
from jax.experimental.pallas import tpu as pltpu
from jax.experimental.pallas import tpu_sc as plsc

DIM = 64
N_POS = 5
N_CTX = 25
S = 32
NBUF = 2
NC = 2
NS = 16
NW = NC * NS
LANES = 16
NIDX = (1 + N_CTX) * S
SLAB = 1024


def _sc_dots(v_bf, u_bf, idx_slabs, B):
    per_w = B // NW
    n_chunks = per_w // S

    mesh = plsc.VectorSubcoreMesh(core_axis_name="c", subcore_axis_name="s")

    @functools.partial(
        pl.kernel,
        mesh=mesh,
        compiler_params=pltpu.CompilerParams(
            use_tc_tiling_on_sc=False, needs_layout_passes=False),
        out_type=jax.ShapeDtypeStruct((B // 4, 128), jnp.float32),
        scratch_types=[
            pltpu.VMEM((NBUF, SLAB), jnp.int32),
            pltpu.VMEM((NBUF, S, DIM), jnp.bfloat16),
            pltpu.VMEM((NBUF, N_CTX, S, DIM), jnp.bfloat16),
            pltpu.VMEM((128, 128), jnp.float32),
            pltpu.SemaphoreType.DMA,
            pltpu.SemaphoreType.DMA,
        ],
    )
    def sc_kernel(v_hbm, u_hbm, idx_hbm, out_hbm,
                  idx_v, v_buf, u_buf, dots, sg0, sg1):
        wid = lax.axis_index("s") * NC + lax.axis_index("c")
        sgs = (sg0, sg1)

        def issue(chunk, b):
            slab = (wid * n_chunks + chunk) * SLAB
            pltpu.sync_copy(idx_hbm.at[pl.ds(slab, SLAB)], idx_v.at[b])
            pltpu.async_copy(v_hbm.at[idx_v.at[b, pl.ds(0, S)]],
                             v_buf.at[b], sgs[b])
            for j in range(N_CTX):
                pltpu.async_copy(u_hbm.at[idx_v.at[b, pl.ds((1 + j) * S, S)]],
                                 u_buf.at[b, j], sgs[b])

        def drain_gathers(b):
            pltpu.make_async_copy(v_hbm.at[idx_v.at[b, pl.ds(0, S)]],
                                  v_buf.at[b], sgs[b]).wait()
            for j in range(N_CTX):
                pltpu.make_async_copy(
                    u_hbm.at[idx_v.at[b, pl.ds((1 + j) * S, S)]],
                    u_buf.at[b, j], sgs[b]).wait()

        def compute(b, chunk):
            def body_s(s, carry):
                c = []
                for t in range(2):
                    ch = v_buf[b, s, pl.ds(t * 2 * LANES, 2 * LANES)]
                    c.extend(plsc.unpack(ch, format=plsc.PackFormat.INTERLEAVED))
                lane = lax.iota(jnp.int32, LANES)
                dlo = jnp.zeros((LANES,), jnp.float32)
                dhi = jnp.full((LANES,), 30.0, jnp.float32)
                for j in range(N_CTX):
                    u = []
                    for t in range(2):
                        uh = u_buf[b, j, s, pl.ds(t * 2 * LANES, 2 * LANES)]
                        u.extend(
                            plsc.unpack(uh, format=plsc.PackFormat.INTERLEAVED))
                    acc = u[0] * c[0]
                    for t in range(1, 4):
                        acc = acc + u[t] * c[t]
                    d = jnp.sum(acc)
                    d = d if j < N_POS else -d
                    if j < LANES:
                        dlo = jnp.where(lane == j, d, dlo)
                    else:
                        dhi = jnp.where(lane == (j - LANES), d, dhi)
                g = chunk * S + s
                row = g >> 2
                lb = (g & 3) * 2 * LANES
                dots[row, pl.ds(lb, LANES)] = dlo
                dots[row, pl.ds(lb + LANES, LANES)] = dhi
                return carry
            lax.fori_loop(0, S, body_s, 0)

        issue(0, 0)

        def outer(i, carry):
            for b in range(NBUF):
                chunk = NBUF * i + b
                nb = 1 - b

                @pl.when(chunk + 1 < n_chunks)
                def _():
                    issue(chunk + 1, nb)

                drain_gathers(b)
                compute(b, chunk)
            return carry

        lax.fori_loop(0, n_chunks // NBUF, outer, 0)
        pltpu.sync_copy(dots, out_hbm.at[pl.ds(wid * (per_w // 4), per_w // 4)])

    return sc_kernel(v_bf, u_bf, idx_slabs)


def _loss_body(dots_ref, out_ref):
    x = dots_ref[...]
    y = jax.nn.log_sigmoid(x)
    col = lax.broadcasted_iota(jnp.int32, (128, 4), 0)
    grp = lax.broadcasted_iota(jnp.int32, (128, 4), 1)
    sel = ((col // 32) == grp).astype(jnp.float32)
    out_ref[...] = -lax.dot(y, sel, precision=lax.Precision.HIGHEST)


def kernel(center_words, positive_words, negative_words, v_weight, u_weight):
    B = center_words.shape[0]
    all_idx = jnp.concatenate(
        [center_words[None, :], positive_words.T, negative_words.T], axis=0)
    all_idx = all_idx.astype(jnp.int32)

    n_chunks = B // NW // S
    slabs = all_idx.reshape(1 + N_CTX, NW, n_chunks, S)
    slabs = slabs.transpose(1, 2, 0, 3).reshape(NW * n_chunks, NIDX)
    slabs = jnp.pad(slabs, ((0, 0), (0, SLAB - NIDX)))
    idx_slabs = slabs.reshape(-1)

    v_bf = v_weight.astype(jnp.bfloat16)
    u_bf = u_weight.astype(jnp.bfloat16)

    dots = _sc_dots(v_bf, u_bf, idx_slabs, B)

    bt = 1024
    loss4 = pl.pallas_call(
        _loss_body,
        grid=(B // 4 // bt,),
        in_specs=[pl.BlockSpec((bt, 128), lambda i: (i, 0))],
        out_specs=pl.BlockSpec((bt, 4), lambda i: (i, 0)),
        out_shape=jax.ShapeDtypeStruct((B // 4, 4), jnp.float32),
    )(dots)
    return loss4.reshape(B)
